# baseline (speedup 1.0000x reference)
import functools

import jax
import jax.numpy as jnp
from jax import lax
from jax.experimental import pallas as pl
from jax.experimental.pallas import tpu as pltpu
from jax.experimental.pallas import tpu_sc as plsc

B = 4096
NW = 32
BPW = B // NW
NBLK = 8
BLK = B // NBLK

_FEATS = [
    ('uid', 32, 's'),
    ('gender', 16, 's'),
    ('birthyear', 16, 's'),
    ('constellation', 16, 's'),
    ('character_tag', 32, 'b'),
    ('user_tag', 32, 'b'),
    ('theme_tag', 32, 'b'),
    ('pid', 32, 's'),
    ('title', 32, 's'),
    ('activity_type', 16, 's'),
    ('activity_theme_tag', 32, 'b'),
    ('people_max', 16, 's'),
    ('people_min', 16, 's'),
    ('sex_limit_type', 16, 'm'),
    ('longitude', 16, 'm'),
    ('latitude', 16, 's'),
    ('pre_amt', 16, 's'),
    ('price_type', 16, 's'),
]
_NF = len(_FEATS)
_N_USER = 7

_FEAT_ROW = {}
_COL_OFF = []
_r = 0
_c = 0
for _name, _d, _k in _FEATS:
    _FEAT_ROW[_name] = _r
    _COL_OFF.append(_c)
    _r += 1 if _k == 's' else 5
    _c += _d
_N_IDX_ROWS = _r
_D_IN = _c
_D_H = 32 * _NF
_D_CMB = 512


def _sc_gather(idx_all, tables):
    mesh = plsc.VectorSubcoreMesh(core_axis_name="c", subcore_axis_name="s")

    @functools.partial(
        pl.kernel,
        out_type=jax.ShapeDtypeStruct((B, _D_IN), jnp.float32),
        mesh=mesh,
        compiler_params=pltpu.CompilerParams(use_tc_tiling_on_sc=False),
        scratch_types=[
            pltpu.VMEM((_N_IDX_ROWS, BPW), jnp.int32),
            pltpu.VMEM((5 * BPW, 32), jnp.float32),
            pltpu.VMEM((5 * BPW, 16), jnp.float32),
            pltpu.VMEM((BPW, 32), jnp.float32),
            pltpu.VMEM((BPW, 16), jnp.float32),
            pltpu.SemaphoreType.DMA,
        ],
    )
    def body(idx_hbm, *rest):
        tabs = rest[:_NF]
        out = rest[_NF]
        idx_v, rows32, rows16, red32, red16, sem = rest[_NF + 1:]
        wid = lax.axis_index("s") * 2 + lax.axis_index("c")
        base = wid * BPW
        pltpu.sync_copy(idx_hbm.at[:, pl.ds(base, BPW)], idx_v)
        for fi, (name, d, kind) in enumerate(_FEATS):
            r0 = _FEAT_ROW[name]
            c0 = _COL_OFF[fi]
            rows = rows32 if d == 32 else rows16
            red = red32 if d == 32 else red16
            dst = out.at[pl.ds(base, BPW), pl.ds(c0, d)]
            if kind == 's':
                pltpu.async_copy(
                    tabs[fi].at[idx_v.at[r0]], rows.at[pl.ds(0, BPW)],
                    sem).wait()
                pltpu.sync_copy(rows.at[pl.ds(0, BPW)], dst)
            else:
                cps = [
                    pltpu.async_copy(
                        tabs[fi].at[idx_v.at[r0 + j]],
                        rows.at[pl.ds(j * BPW, BPW)], sem)
                    for j in range(5)
                ]
                for cp in cps:
                    cp.wait()
                nch = d // 16

                def red_body(r, carry, rows=rows, red=red, nch=nch):
                    for ch in range(nch):
                        acc = rows[r, pl.ds(ch * 16, 16)]
                        for j in range(1, 5):
                            acc = acc + rows[j * BPW + r, pl.ds(ch * 16, 16)]
                        red[r, pl.ds(ch * 16, 16)] = acc
                    return carry

                lax.fori_loop(0, BPW, red_body, 0)
                pltpu.sync_copy(red, dst)

    return body(idx_all, *tables)


def _k1_body(x_ref, w_ref, b_ref, r_ref, cs_ref, cq_ref):
    h = lax.dot_general(
        x_ref[...], w_ref[...], (((1,), (0,)), ((), ())),
        preferred_element_type=jnp.float32,
        precision=lax.Precision.HIGHEST) + b_ref[...]
    r = jnp.maximum(h, 0.0)
    r_ref[...] = r
    cs_ref[...] = jnp.sum(r, axis=0, keepdims=True).reshape(1, 1, _D_H)
    cq_ref[...] = jnp.sum(r * r, axis=0, keepdims=True).reshape(1, 1, _D_H)


def _k2_body(r_ref, cs_ref, cq_ref, wc_ref, bc_ref, gb_ref, out_ref):
    cs = cs_ref[...].reshape(NBLK, _D_H)
    cq = cq_ref[...].reshape(NBLK, _D_H)
    gamma = gb_ref[0, 0]
    beta = gb_ref[0, 1]
    n = float(B * 32)
    scales = []
    shifts = []
    for f in range(_NF):
        s = jnp.sum(cs[:, f * 32:(f + 1) * 32])
        q = jnp.sum(cq[:, f * 32:(f + 1) * 32])
        m = s / n
        v = q / n - m * m
        sc = gamma * lax.rsqrt(v + 1e-5)
        scales.append(jnp.full((1, 32), sc, jnp.float32))
        shifts.append(jnp.full((1, 32), beta - m * sc, jnp.float32))
    scale = jnp.concatenate(scales, axis=1)
    shift = jnp.concatenate(shifts, axis=1)
    feats = r_ref[...] * scale + shift
    up = jnp.tanh(
        lax.dot_general(
            feats, wc_ref[...], (((1,), (0,)), ((), ())),
            preferred_element_type=jnp.float32,
            precision=lax.Precision.HIGHEST) + bc_ref[...])
    out_ref[...] = jnp.sum(
        up[:, :_D_CMB // 2] * up[:, _D_CMB // 2:], axis=1, keepdims=True)


def _tc_dense(x, params):
    w_all = jnp.zeros((_D_IN, _D_H), jnp.float32)
    b_all = []
    for fi, (name, d, kind) in enumerate(_FEATS):
        w = params['fc_' + name + '_W']
        if kind == 'm':
            w = w * 0.2
        w_all = w_all.at[_COL_OFF[fi]:_COL_OFF[fi] + d,
                         fi * 32:(fi + 1) * 32].set(w)
        b_all.append(params['fc_' + name + '_b'])
    b_all = jnp.concatenate(b_all).reshape(1, _D_H)

    r, cs, cq = pl.pallas_call(
        _k1_body,
        grid=(NBLK,),
        in_specs=[
            pl.BlockSpec((BLK, _D_IN), lambda i: (i, 0)),
            pl.BlockSpec((_D_IN, _D_H), lambda i: (0, 0)),
            pl.BlockSpec((1, _D_H), lambda i: (0, 0)),
        ],
        out_specs=[
            pl.BlockSpec((BLK, _D_H), lambda i: (i, 0)),
            pl.BlockSpec((1, 1, _D_H), lambda i: (i, 0, 0)),
            pl.BlockSpec((1, 1, _D_H), lambda i: (i, 0, 0)),
        ],
        out_shape=[
            jax.ShapeDtypeStruct((B, _D_H), jnp.float32),
            jax.ShapeDtypeStruct((NBLK, 1, _D_H), jnp.float32),
            jax.ShapeDtypeStruct((NBLK, 1, _D_H), jnp.float32),
        ],
    )(x, w_all, b_all)

    nu = _N_USER * 32
    wc = jnp.zeros((_D_H, _D_CMB), jnp.float32)
    wc = wc.at[:nu, 0:200].set(params['fc_user_combine_W'])
    wc = wc.at[nu:, _D_CMB // 2:_D_CMB // 2 + 200].set(
        params['fc_party_combine_W'])
    bc = jnp.zeros((1, _D_CMB), jnp.float32)
    bc = bc.at[0, 0:200].set(params['fc_user_combine_b'])
    bc = bc.at[0, _D_CMB // 2:_D_CMB // 2 + 200].set(
        params['fc_party_combine_b'])
    gb = jnp.stack([params['bn_gamma'], params['bn_beta']]).reshape(1, 2)

    rating = pl.pallas_call(
        _k2_body,
        grid=(NBLK,),
        in_specs=[
            pl.BlockSpec((BLK, _D_H), lambda i: (i, 0)),
            pl.BlockSpec((NBLK, 1, _D_H), lambda i: (0, 0, 0)),
            pl.BlockSpec((NBLK, 1, _D_H), lambda i: (0, 0, 0)),
            pl.BlockSpec((_D_H, _D_CMB), lambda i: (0, 0)),
            pl.BlockSpec((1, _D_CMB), lambda i: (0, 0)),
            pl.BlockSpec((1, 2), lambda i: (0, 0)),
        ],
        out_specs=pl.BlockSpec((BLK, 1), lambda i: (i, 0)),
        out_shape=jax.ShapeDtypeStruct((B, 1), jnp.float32),
    )(r, cs, cq, wc, bc, gb)
    return rating.reshape(B)


def kernel(uid, gender, birthyear, constellation, character_tag, user_tag,
           theme_tag, pid, title, activity_type, activity_theme_tag,
           people_max, people_min, sex_limit_type, longitude, latitude,
           pre_amt, price_type, params):
    idx = {
        'uid': uid, 'gender': gender, 'birthyear': birthyear,
        'constellation': constellation, 'character_tag': character_tag,
        'user_tag': user_tag, 'theme_tag': theme_tag, 'pid': pid,
        'title': title, 'activity_type': activity_type,
        'activity_theme_tag': activity_theme_tag, 'people_max': people_max,
        'people_min': people_min, 'sex_limit_type': sex_limit_type,
        'longitude': longitude, 'latitude': latitude, 'pre_amt': pre_amt,
        'price_type': price_type,
    }
    rows = []
    for name, d, kind in _FEATS:
        a = idx[name].astype(jnp.int32)
        rows.append(a.reshape(1, B) if kind == 's' else a.T)
    idx_all = jnp.concatenate(rows, axis=0)
    tables = [params['emb_' + n] for n, _, _ in _FEATS]
    x = _sc_gather(idx_all, tables)
    return _tc_dense(x, params)


# speedup vs baseline: 3.7077x; 3.7077x over previous
import functools

import jax
import jax.numpy as jnp
from jax import lax
from jax.experimental import pallas as pl
from jax.experimental.pallas import tpu as pltpu
from jax.experimental.pallas import tpu_sc as plsc

B = 4096
NW = 32
BPW = B // NW
NBLK = 8
BLK = B // NBLK

_FEATS = [
    ('uid', 32, 's'),
    ('gender', 16, 's'),
    ('birthyear', 16, 's'),
    ('constellation', 16, 's'),
    ('character_tag', 32, 'b'),
    ('user_tag', 32, 'b'),
    ('theme_tag', 32, 'b'),
    ('pid', 32, 's'),
    ('title', 32, 's'),
    ('activity_type', 16, 's'),
    ('activity_theme_tag', 32, 'b'),
    ('people_max', 16, 's'),
    ('people_min', 16, 's'),
    ('sex_limit_type', 16, 'm'),
    ('longitude', 16, 'm'),
    ('latitude', 16, 's'),
    ('pre_amt', 16, 's'),
    ('price_type', 16, 's'),
]
_NF = len(_FEATS)
_N_USER = 7

_BIG = ['uid', 'pid', 'pre_amt']
_BIG_ROW = {'uid': 0, 'pid': 32, 'pre_amt': 64}
_D_BIG = 80

_FEAT_ROW = {}
_r = 0
for _name, _d, _k in _FEATS:
    if _name in _BIG:
        continue
    _FEAT_ROW[_name] = _r
    _r += 1 if _k == 's' else 5
_N_IDX_ROWS = _r

_XCOL = {}
_c = 0
for _name, _d, _k in _FEATS:
    if _name not in _BIG:
        _XCOL[_name] = _c
        _c += _d
_D_X = _c
_D_H = 32 * _NF
_D_CMB = 512

_CI = {}
for _name, _d, _k in _FEATS:
    if _name not in _BIG:
        _CI[_name] = len(_CI)
_SINGLES = [(_CI[n], n, d) for n, d, k in _FEATS
            if k == 's' and n not in _BIG]
_BAGS = [(_CI[n], n, d) for n, d, k in _FEATS if k != 's']
_BAGS_EARLY = [_BAGS[0], _BAGS[1], _BAGS[4], _BAGS[5]]
_BAGS_LATE = [(_BAGS[2], 0), (_BAGS[3], 1)]


def _sc_gather(idx_all, tables):
    mesh = plsc.VectorSubcoreMesh(core_axis_name="c", subcore_axis_name="s")

    scratch = [pltpu.VMEM((_N_IDX_ROWS, BPW), jnp.int32)]
    for _, n, d in _SINGLES:
        scratch.append(pltpu.VMEM((BPW, d), jnp.float32))
    for _, _, d in _BAGS_EARLY:
        scratch.append(pltpu.VMEM((5 * BPW, d), jnp.float32))
    scratch += [pltpu.SemaphoreType.DMA, pltpu.SemaphoreType.DMA,
                pltpu.SemaphoreType.DMA, pltpu.SemaphoreType.DMA,
                pltpu.SemaphoreType.DMA]

    @functools.partial(
        pl.kernel,
        out_type=jax.ShapeDtypeStruct((B, _D_X), jnp.float32),
        mesh=mesh,
        compiler_params=pltpu.CompilerParams(use_tc_tiling_on_sc=False),
        scratch_types=scratch,
    )
    def body(idx_hbm, *rest):
        tabs = rest[:len(tables)]
        out = rest[len(tables)]
        sc = list(rest[len(tables) + 1:])
        idx_v = sc[0]
        sbufs = sc[1:1 + len(_SINGLES)]
        bbufs = sc[1 + len(_SINGLES):1 + len(_SINGLES) + len(_BAGS_EARLY)]
        semg, sems, semw = sc[-5], sc[-4], sc[-3]
        semw_b0, semw_b1 = sc[-2], sc[-1]
        wid = lax.axis_index("s") * 2 + lax.axis_index("c")
        base = wid * BPW
        pltpu.sync_copy(idx_hbm.at[:, pl.ds(base, BPW)], idx_v)

        def fire_bag(bag, buf):
            fi, name, d = bag
            r0 = _FEAT_ROW[name]
            return [
                pltpu.async_copy(
                    tabs[fi].at[idx_v.at[r0 + j]],
                    buf.at[pl.ds(j * BPW, BPW)], semg)
                for j in range(5)
            ]

        def reduce_bag(bag, buf):
            nch = bag[2] // 16

            def red_body(r, carry):
                for ch in range(nch):
                    acc = buf[r, pl.ds(ch * 16, 16)]
                    for j in range(1, 5):
                        acc = acc + buf[j * BPW + r, pl.ds(ch * 16, 16)]
                    buf[r, pl.ds(ch * 16, 16)] = acc
                return carry

            lax.fori_loop(0, BPW, red_body, 0)

        def writeback(name, d, buf, sem=None):
            return pltpu.async_copy(
                buf.at[pl.ds(0, BPW)],
                out.at[pl.ds(base, BPW), pl.ds(_XCOL[name], d)],
                semw if sem is None else sem)

        bag_pend = []
        for bag, buf in zip(_BAGS_EARLY, bbufs):
            bag_pend += fire_bag(bag, buf)
        s_pend = [
            pltpu.async_copy(tabs[fi].at[idx_v.at[_FEAT_ROW[n]]], buf, sems)
            for (fi, n, d), buf in zip(_SINGLES, sbufs)
        ]
        for cp in s_pend:
            cp.wait()
        wb = [writeback(n, d, buf)
              for (fi, n, d), buf in zip(_SINGLES, sbufs)]
        for cp in bag_pend:
            cp.wait()
        late_pend = []
        for (bag, bi), bsem in zip(_BAGS_LATE, (semw_b0, semw_b1)):
            early = _BAGS_EARLY[bi]
            reduce_bag(early, bbufs[bi])
            w = writeback(early[1], early[2], bbufs[bi], bsem)
            w.wait()
            late_pend.append(fire_bag(bag, bbufs[bi]))
        for i, (bag, buf) in enumerate(zip(_BAGS_EARLY, bbufs)):
            if i in (0, 1):
                continue
            reduce_bag(bag, buf)
            wb.append(writeback(bag[1], bag[2], buf))
        for (bag, bi), pend in zip(_BAGS_LATE, late_pend):
            for cp in pend:
                cp.wait()
            reduce_bag(bag, bbufs[bi])
            wb.append(writeback(bag[1], bag[2], bbufs[bi]))
        for cp in wb:
            cp.wait()

    return body(idx_all, *tables)


def _sc_pack_big(idx3, tab_u, tab_p, tab_r):
    mesh = plsc.VectorSubcoreMesh(core_axis_name="c", subcore_axis_name="s")

    @functools.partial(
        pl.kernel,
        out_type=jax.ShapeDtypeStruct((_D_BIG, B), jnp.float32),
        mesh=mesh,
        compiler_params=pltpu.CompilerParams(
            use_tc_tiling_on_sc=True, needs_layout_passes=False),
        scratch_types=[
            pltpu.VMEM((3, BPW), jnp.int32),
            pltpu.VMEM((_D_BIG, BPW), jnp.float32),
            pltpu.VMEM((32 * 16, 128), jnp.float32),
            pltpu.SemaphoreType.DMA,
            pltpu.SemaphoreType.DMA,
        ],
    )
    def body(idx_hbm, tu, tp, tr, out, idx_v, buf, stage, semc, semw):
        wid = lax.axis_index("s") * 2 + lax.axis_index("c")
        base = wid * BPW
        pltpu.sync_copy(idx_hbm.at[:, pl.ds(base, BPW)], idx_v)
        iota = jax.lax.iota(jnp.int32, 16)

        for t, (tab, r0, d) in enumerate(
                ((tu, 0, 32), (tp, 32, 32), (tr, 64, 16))):

            def grp(g, carry, tab=tab, r0=r0, d=d, t=t):
                vec = idx_v[t, pl.ds(g * 16, 16)]
                svals = []
                pend = []
                for l in range(16):
                    s = jnp.sum(jnp.where(iota == l, vec, 0))
                    j = pl.multiple_of((s // 128) * 128, 128)
                    svals.append(s - j)
                    pend.append(pltpu.async_copy(
                        tab.at[:, pl.ds(j, 128)],
                        stage.at[pl.ds(l * 32, d)], semc))
                for cp in pend:
                    cp.wait()
                for l in range(16):
                    col = jnp.full((16,), svals[l], jnp.int32)
                    dstc = jnp.full((16,), g * 16 + l, jnp.int32)
                    for half in range(d // 16):
                        rows = iota + (l * 32 + half * 16)
                        v = plsc.load_gather(stage, [rows, col])
                        plsc.store_scatter(
                            buf, [iota + (r0 + half * 16), dstc], v)
                return carry

            lax.fori_loop(0, BPW // 16, grp, 0)
        pltpu.async_copy(buf, out.at[:, pl.ds(base, BPW)], semw).wait()

    return body(idx3, tab_u, tab_p, tab_r)


def _k1_body(x_ref, xb_ref, wc_ref, wb_ref, b_ref, r_ref, cs_ref, cq_ref):
    h = lax.dot_general(
        x_ref[...], wc_ref[...], (((1,), (0,)), ((), ())),
        preferred_element_type=jnp.float32,
        precision=lax.Precision.HIGHEST)
    h = h + lax.dot_general(
        xb_ref[...], wb_ref[...], (((0,), (0,)), ((), ())),
        preferred_element_type=jnp.float32,
        precision=lax.Precision.HIGHEST)
    h = h + b_ref[...]
    r = jnp.maximum(h, 0.0)
    r_ref[...] = r
    cs_ref[...] = jnp.sum(r, axis=0, keepdims=True).reshape(1, 1, _D_H)
    cq_ref[...] = jnp.sum(r * r, axis=0, keepdims=True).reshape(1, 1, _D_H)


def _k2_body(r_ref, cs_ref, cq_ref, wc_ref, bc_ref, gb_ref, out_ref):
    cs = cs_ref[...].reshape(NBLK, _D_H)
    cq = cq_ref[...].reshape(NBLK, _D_H)
    gamma = gb_ref[0, 0]
    beta = gb_ref[0, 1]
    n = float(B * 32)
    scales = []
    shifts = []
    for f in range(_NF):
        s = jnp.sum(cs[:, f * 32:(f + 1) * 32])
        q = jnp.sum(cq[:, f * 32:(f + 1) * 32])
        m = s / n
        v = q / n - m * m
        sc = gamma * lax.rsqrt(v + 1e-5)
        scales.append(jnp.full((1, 32), sc, jnp.float32))
        shifts.append(jnp.full((1, 32), beta - m * sc, jnp.float32))
    scale = jnp.concatenate(scales, axis=1)
    shift = jnp.concatenate(shifts, axis=1)
    feats = r_ref[...] * scale + shift
    up = jnp.tanh(
        lax.dot_general(
            feats, wc_ref[...], (((1,), (0,)), ((), ())),
            preferred_element_type=jnp.float32,
            precision=lax.Precision.HIGHEST) + bc_ref[...])
    out_ref[...] = jnp.sum(
        up[:, :_D_CMB // 2] * up[:, _D_CMB // 2:], axis=1, keepdims=True)


def _tc_dense(x, xbt, params):
    w_c = jnp.zeros((_D_X, _D_H), jnp.float32)
    w_b = jnp.zeros((_D_BIG, _D_H), jnp.float32)
    b_all = []
    for fi, (name, d, kind) in enumerate(_FEATS):
        w = params['fc_' + name + '_W']
        if kind == 'm':
            w = w * 0.2
        if name in _BIG:
            r0 = _BIG_ROW[name]
            w_b = w_b.at[r0:r0 + d, fi * 32:(fi + 1) * 32].set(w)
        else:
            c0 = _XCOL[name]
            w_c = w_c.at[c0:c0 + d, fi * 32:(fi + 1) * 32].set(w)
        b_all.append(params['fc_' + name + '_b'])
    b_all = jnp.concatenate(b_all).reshape(1, _D_H)

    r, cs, cq = pl.pallas_call(
        _k1_body,
        grid=(NBLK,),
        in_specs=[
            pl.BlockSpec((BLK, _D_X), lambda i: (i, 0)),
            pl.BlockSpec((_D_BIG, BLK), lambda i: (0, i)),
            pl.BlockSpec((_D_X, _D_H), lambda i: (0, 0)),
            pl.BlockSpec((_D_BIG, _D_H), lambda i: (0, 0)),
            pl.BlockSpec((1, _D_H), lambda i: (0, 0)),
        ],
        out_specs=[
            pl.BlockSpec((BLK, _D_H), lambda i: (i, 0)),
            pl.BlockSpec((1, 1, _D_H), lambda i: (i, 0, 0)),
            pl.BlockSpec((1, 1, _D_H), lambda i: (i, 0, 0)),
        ],
        out_shape=[
            jax.ShapeDtypeStruct((B, _D_H), jnp.float32),
            jax.ShapeDtypeStruct((NBLK, 1, _D_H), jnp.float32),
            jax.ShapeDtypeStruct((NBLK, 1, _D_H), jnp.float32),
        ],
    )(x, xbt, w_c, w_b, b_all)

    nu = _N_USER * 32
    wc = jnp.zeros((_D_H, _D_CMB), jnp.float32)
    wc = wc.at[:nu, 0:200].set(params['fc_user_combine_W'])
    wc = wc.at[nu:, _D_CMB // 2:_D_CMB // 2 + 200].set(
        params['fc_party_combine_W'])
    bc = jnp.zeros((1, _D_CMB), jnp.float32)
    bc = bc.at[0, 0:200].set(params['fc_user_combine_b'])
    bc = bc.at[0, _D_CMB // 2:_D_CMB // 2 + 200].set(
        params['fc_party_combine_b'])
    gb = jnp.stack([params['bn_gamma'], params['bn_beta']]).reshape(1, 2)

    rating = pl.pallas_call(
        _k2_body,
        grid=(NBLK,),
        in_specs=[
            pl.BlockSpec((BLK, _D_H), lambda i: (i, 0)),
            pl.BlockSpec((NBLK, 1, _D_H), lambda i: (0, 0, 0)),
            pl.BlockSpec((NBLK, 1, _D_H), lambda i: (0, 0, 0)),
            pl.BlockSpec((_D_H, _D_CMB), lambda i: (0, 0)),
            pl.BlockSpec((1, _D_CMB), lambda i: (0, 0)),
            pl.BlockSpec((1, 2), lambda i: (0, 0)),
        ],
        out_specs=pl.BlockSpec((BLK, 1), lambda i: (i, 0)),
        out_shape=jax.ShapeDtypeStruct((B, 1), jnp.float32),
    )(r, cs, cq, wc, bc, gb)
    return rating.reshape(B)


def kernel(uid, gender, birthyear, constellation, character_tag, user_tag,
           theme_tag, pid, title, activity_type, activity_theme_tag,
           people_max, people_min, sex_limit_type, longitude, latitude,
           pre_amt, price_type, params):
    idx = {
        'uid': uid, 'gender': gender, 'birthyear': birthyear,
        'constellation': constellation, 'character_tag': character_tag,
        'user_tag': user_tag, 'theme_tag': theme_tag, 'pid': pid,
        'title': title, 'activity_type': activity_type,
        'activity_theme_tag': activity_theme_tag, 'people_max': people_max,
        'people_min': people_min, 'sex_limit_type': sex_limit_type,
        'longitude': longitude, 'latitude': latitude, 'pre_amt': pre_amt,
        'price_type': price_type,
    }
    rows = []
    tables = []
    for name, d, kind in _FEATS:
        if name in _BIG:
            continue
        a = idx[name].astype(jnp.int32)
        rows.append(a.reshape(1, B) if kind == 's' else a.T)
        tables.append(params['emb_' + name])
    idx_all = jnp.concatenate(rows, axis=0)
    idx3 = jnp.stack([idx[n].astype(jnp.int32) for n in _BIG])
    x = _sc_gather(idx_all, tables)
    xbt = _sc_pack_big(idx3, params['emb_uid'].T, params['emb_pid'].T,
                       params['emb_pre_amt'].T)
    return _tc_dense(x, xbt, params)
